# baseline (speedup 1.0000x reference)
import jax
import jax.numpy as jnp
from jax.experimental import pallas as pl

TEXT_DIM = 512
NUM_CLASSES = 20
KPAD = 32
OUT_H = 128
OUT_W = 256
IN_W = 512


def _emb_kernel(seg_ref, et_ref, sel_ref, out_ref):
    row = seg_ref[0, 0, 0:1, :]
    kio = jax.lax.broadcasted_iota(jnp.int32, (KPAD, IN_W), 0)
    oh = (kio == row).astype(jnp.float32)
    oh_ds = jax.lax.dot_general(
        oh, sel_ref[...], (((1,), (0,)), ((), ())),
        preferred_element_type=jnp.float32)
    res = jax.lax.dot_general(
        et_ref[...], oh_ds, (((1,), (0,)), ((), ())),
        preferred_element_type=jnp.float32)
    out_ref[0] = res


@jax.jit
def kernel(seg_map, class_embeddings):
    bs = seg_map.shape[0]
    seg_r = seg_map.reshape(bs, OUT_H, 2, IN_W)
    et = jnp.zeros((TEXT_DIM, KPAD), jnp.float32)
    et = et.at[:, :NUM_CLASSES].set(class_embeddings.T)
    sel = (jax.lax.broadcasted_iota(jnp.int32, (IN_W, OUT_W), 0)
           == 2 * jax.lax.broadcasted_iota(jnp.int32, (IN_W, OUT_W), 1)
           ).astype(jnp.float32)
    grid = (bs, OUT_H)
    out = pl.pallas_call(
        _emb_kernel,
        grid=grid,
        in_specs=[
            pl.BlockSpec((1, 1, 2, IN_W), lambda b, i: (b, i, 0, 0)),
            pl.BlockSpec((TEXT_DIM, KPAD), lambda b, i: (0, 0)),
            pl.BlockSpec((IN_W, OUT_W), lambda b, i: (0, 0)),
        ],
        out_specs=pl.BlockSpec((1, TEXT_DIM, OUT_W), lambda b, i: (b, 0, i)),
        out_shape=jax.ShapeDtypeStruct((bs, TEXT_DIM, OUT_H * OUT_W), jnp.float32),
    )(seg_r, et, sel)
    return out.reshape(bs, TEXT_DIM, OUT_H, OUT_W)


# speedup vs baseline: 1.6676x; 1.6676x over previous
import jax
import jax.numpy as jnp
from jax.experimental import pallas as pl

TEXT_DIM = 512
NUM_CLASSES = 20
KPAD = 32
OUT_H = 128
OUT_W = 256
IN_W = 512
ROWS_PER_STEP = 8


def _emb_kernel(seg_ref, et_ref, sel_ref, out_ref):
    et = et_ref[...]
    sel = sel_ref[...]
    kio = jax.lax.broadcasted_iota(jnp.int32, (KPAD, IN_W), 0)
    for r in range(ROWS_PER_STEP):
        row = seg_ref[0, r, 0:1, :]
        oh = (kio == row).astype(jnp.float32)
        oh_ds = jax.lax.dot_general(
            oh, sel, (((1,), (0,)), ((), ())),
            preferred_element_type=jnp.float32)
        res = jax.lax.dot_general(
            et, oh_ds, (((1,), (0,)), ((), ())),
            preferred_element_type=jnp.float32)
        out_ref[0, :, r * OUT_W:(r + 1) * OUT_W] = res


@jax.jit
def kernel(seg_map, class_embeddings):
    bs = seg_map.shape[0]
    seg_r = seg_map.reshape(bs, OUT_H, 2, IN_W)
    et = jnp.zeros((TEXT_DIM, KPAD), jnp.float32)
    et = et.at[:, :NUM_CLASSES].set(class_embeddings.T)
    sel = (jax.lax.broadcasted_iota(jnp.int32, (IN_W, OUT_W), 0)
           == 2 * jax.lax.broadcasted_iota(jnp.int32, (IN_W, OUT_W), 1)
           ).astype(jnp.float32)
    grid = (bs, OUT_H // ROWS_PER_STEP)
    out = pl.pallas_call(
        _emb_kernel,
        grid=grid,
        in_specs=[
            pl.BlockSpec((1, ROWS_PER_STEP, 2, IN_W), lambda b, i: (b, i, 0, 0)),
            pl.BlockSpec((TEXT_DIM, KPAD), lambda b, i: (0, 0)),
            pl.BlockSpec((IN_W, OUT_W), lambda b, i: (0, 0)),
        ],
        out_specs=pl.BlockSpec(
            (1, TEXT_DIM, ROWS_PER_STEP * OUT_W), lambda b, i: (b, 0, i)),
        out_shape=jax.ShapeDtypeStruct(
            (bs, TEXT_DIM, OUT_H * OUT_W), jnp.float32),
    )(seg_r, et, sel)
    return out.reshape(bs, TEXT_DIM, OUT_H, OUT_W)


# speedup vs baseline: 1.7852x; 1.0705x over previous
import jax
import jax.numpy as jnp
from jax.experimental import pallas as pl

TEXT_DIM = 512
NUM_CLASSES = 20
KPAD = 32
OUT_H = 128
OUT_W = 256
IN_W = 512
ROWS_PER_STEP = 8


def _emb_kernel(seg_ref, et_ref, sel_ref, out_ref):
    rows = seg_ref[0, :, 0:IN_W].astype(jnp.float32)
    rowds = jax.lax.dot_general(
        rows, sel_ref[...], (((1,), (0,)), ((), ())),
        preferred_element_type=jnp.float32)
    rid = rowds.astype(jnp.int32)
    kio = jax.lax.broadcasted_iota(jnp.int32, (KPAD, OUT_W), 0)
    oh = jnp.concatenate(
        [(kio == rid[r:r + 1, :]) for r in range(ROWS_PER_STEP)],
        axis=1).astype(jnp.float32)
    out_ref[0, :, :] = jax.lax.dot_general(
        et_ref[...], oh, (((1,), (0,)), ((), ())),
        preferred_element_type=jnp.float32)


@jax.jit
def kernel(seg_map, class_embeddings):
    bs = seg_map.shape[0]
    seg_r = seg_map.reshape(bs, OUT_H, 2 * IN_W)
    et = jnp.zeros((TEXT_DIM, KPAD), jnp.float32)
    et = et.at[:, :NUM_CLASSES].set(class_embeddings.T)
    sel = (jax.lax.broadcasted_iota(jnp.int32, (IN_W, OUT_W), 0)
           == 2 * jax.lax.broadcasted_iota(jnp.int32, (IN_W, OUT_W), 1)
           ).astype(jnp.float32)
    grid = (bs, OUT_H // ROWS_PER_STEP)
    out = pl.pallas_call(
        _emb_kernel,
        grid=grid,
        in_specs=[
            pl.BlockSpec((1, ROWS_PER_STEP, 2 * IN_W), lambda b, i: (b, i, 0)),
            pl.BlockSpec((TEXT_DIM, KPAD), lambda b, i: (0, 0)),
            pl.BlockSpec((IN_W, OUT_W), lambda b, i: (0, 0)),
        ],
        out_specs=pl.BlockSpec(
            (1, TEXT_DIM, ROWS_PER_STEP * OUT_W), lambda b, i: (b, 0, i)),
        out_shape=jax.ShapeDtypeStruct(
            (bs, TEXT_DIM, OUT_H * OUT_W), jnp.float32),
    )(seg_r, et, sel)
    return out.reshape(bs, TEXT_DIM, OUT_H, OUT_W)


# speedup vs baseline: 3.7517x; 2.1016x over previous
import jax
import jax.numpy as jnp
from jax.experimental import pallas as pl

TEXT_DIM = 512
NUM_CLASSES = 20
KPAD = 32
OUT_H = 128
OUT_W = 256
IN_W = 512
ROWS_PER_STEP = 8


def _emb_kernel(seg_ref, lhs_ref, sel_ref, out_ref):
    rows = seg_ref[0, :, 0:IN_W].astype(jnp.float32)
    rowds = jax.lax.dot_general(
        rows, sel_ref[...], (((1,), (0,)), ((), ())),
        preferred_element_type=jnp.float32)
    rid = rowds.astype(jnp.int32)
    kio = jax.lax.broadcasted_iota(jnp.int32, (KPAD * ROWS_PER_STEP, OUT_W), 0)
    krep = jnp.bitwise_and(kio, KPAD - 1)
    rrep = jnp.concatenate(
        [jnp.broadcast_to(rid[r:r + 1, :], (KPAD, OUT_W))
         for r in range(ROWS_PER_STEP)], axis=0)
    oh = (krep == rrep).astype(jnp.float32)
    res = jax.lax.dot_general(
        lhs_ref[...], oh, (((1,), (0,)), ((), ())),
        preferred_element_type=jnp.float32)
    out_ref[0] = res.reshape(TEXT_DIM, ROWS_PER_STEP, OUT_W)


@jax.jit
def kernel(seg_map, class_embeddings):
    bs = seg_map.shape[0]
    seg_r = seg_map.reshape(bs, OUT_H, 2 * IN_W)
    etp = jnp.zeros((TEXT_DIM, KPAD), jnp.float32)
    etp = etp.at[:, :NUM_CLASSES].set(class_embeddings.T)
    eye = jnp.eye(ROWS_PER_STEP, dtype=jnp.float32)
    lhs = (eye[None, :, :, None] * etp[:, None, None, :]).reshape(
        TEXT_DIM * ROWS_PER_STEP, ROWS_PER_STEP * KPAD)
    sel = (jax.lax.broadcasted_iota(jnp.int32, (IN_W, OUT_W), 0)
           == 2 * jax.lax.broadcasted_iota(jnp.int32, (IN_W, OUT_W), 1)
           ).astype(jnp.float32)
    grid = (bs, OUT_H // ROWS_PER_STEP)
    return pl.pallas_call(
        _emb_kernel,
        grid=grid,
        in_specs=[
            pl.BlockSpec((1, ROWS_PER_STEP, 2 * IN_W), lambda b, i: (b, i, 0)),
            pl.BlockSpec((TEXT_DIM * ROWS_PER_STEP, ROWS_PER_STEP * KPAD),
                         lambda b, i: (0, 0)),
            pl.BlockSpec((IN_W, OUT_W), lambda b, i: (0, 0)),
        ],
        out_specs=pl.BlockSpec(
            (1, TEXT_DIM, ROWS_PER_STEP, OUT_W), lambda b, i: (b, 0, i, 0)),
        out_shape=jax.ShapeDtypeStruct(
            (bs, TEXT_DIM, OUT_H, OUT_W), jnp.float32),
    )(seg_r, lhs, sel)


# speedup vs baseline: 3.9741x; 1.0593x over previous
import jax
import jax.numpy as jnp
from jax.experimental import pallas as pl

TEXT_DIM = 512
NUM_CLASSES = 20
KPAD = 32
OUT_H = 128
OUT_W = 256
IN_W = 512
GROUP = 8
BLOCK_ROWS = 32


def _emb_kernel(seg_ref, lhs_ref, sel_ref, out_ref):
    rows = seg_ref[0, :, 0:IN_W].astype(jnp.float32)
    rowds = jax.lax.dot_general(
        rows, sel_ref[...], (((1,), (0,)), ((), ())),
        preferred_element_type=jnp.float32)
    rid = rowds.astype(jnp.int32)
    kio = jax.lax.broadcasted_iota(jnp.int32, (KPAD * GROUP, OUT_W), 0)
    krep = jnp.bitwise_and(kio, KPAD - 1)
    lhs = lhs_ref[...]
    for g in range(BLOCK_ROWS // GROUP):
        rrep = jnp.concatenate(
            [jnp.broadcast_to(rid[g * GROUP + r:g * GROUP + r + 1, :],
                              (KPAD, OUT_W))
             for r in range(GROUP)], axis=0)
        oh = (krep == rrep).astype(jnp.float32)
        res = jax.lax.dot_general(
            lhs, oh, (((1,), (0,)), ((), ())),
            preferred_element_type=jnp.float32)
        out_ref[0, :, g * GROUP:(g + 1) * GROUP, :] = res.reshape(
            TEXT_DIM, GROUP, OUT_W)


@jax.jit
def kernel(seg_map, class_embeddings):
    bs = seg_map.shape[0]
    seg_r = seg_map.reshape(bs, OUT_H, 2 * IN_W)
    etp = jnp.zeros((TEXT_DIM, KPAD), jnp.float32)
    etp = etp.at[:, :NUM_CLASSES].set(class_embeddings.T)
    eye = jnp.eye(GROUP, dtype=jnp.float32)
    lhs = (eye[None, :, :, None] * etp[:, None, None, :]).reshape(
        TEXT_DIM * GROUP, GROUP * KPAD)
    sel = (jax.lax.broadcasted_iota(jnp.int32, (IN_W, OUT_W), 0)
           == 2 * jax.lax.broadcasted_iota(jnp.int32, (IN_W, OUT_W), 1)
           ).astype(jnp.float32)
    grid = (bs, OUT_H // BLOCK_ROWS)
    return pl.pallas_call(
        _emb_kernel,
        grid=grid,
        in_specs=[
            pl.BlockSpec((1, BLOCK_ROWS, 2 * IN_W), lambda b, i: (b, i, 0)),
            pl.BlockSpec((TEXT_DIM * GROUP, GROUP * KPAD),
                         lambda b, i: (0, 0)),
            pl.BlockSpec((IN_W, OUT_W), lambda b, i: (0, 0)),
        ],
        out_specs=pl.BlockSpec(
            (1, TEXT_DIM, BLOCK_ROWS, OUT_W), lambda b, i: (b, 0, i, 0)),
        out_shape=jax.ShapeDtypeStruct(
            (bs, TEXT_DIM, OUT_H, OUT_W), jnp.float32),
    )(seg_r, lhs, sel)
